# baseline (speedup 1.0000x reference)
import functools

import jax
import jax.numpy as jnp
from jax.experimental import pallas as pl
from jax.experimental.pallas import tpu as pltpu

_B = 512
_IT = 512


def _glu_body(meta_ref, tot_ref, xs_ref, wg_ref, wu_ref, wd_ref, aff_ref,
              out_ref, acc_ref, *, ni):
    nb = pl.program_id(0)
    i = pl.program_id(1)

    @pl.when(nb < tot_ref[0])
    def _():
        x = xs_ref[...]
        g = jnp.dot(x, wg_ref[0], preferred_element_type=jnp.float32)
        u = jnp.dot(x, wu_ref[0], preferred_element_type=jnp.float32)
        act = (g * jax.nn.sigmoid(g)) * u
        act = act * aff_ref[0, 0][:, None]
        pd = jnp.dot(act.astype(jnp.bfloat16), wd_ref[0],
                     preferred_element_type=jnp.float32)

        @pl.when(i == 0)
        def _():
            acc_ref[...] = pd

        @pl.when(i > 0)
        def _():
            acc_ref[...] += pd

        @pl.when(i == ni - 1)
        def _():
            out_ref[...] = acc_ref[...]


def _grouped_glu(xs, wg, wu, wd, aff3, block_e, total_nb):
    p, h = xs.shape
    ip = wg.shape[2]
    nb = p // _B
    ni = ip // _IT
    grid = (nb, ni)
    kernel_fn = functools.partial(_glu_body, ni=ni)
    return pl.pallas_call(
        kernel_fn,
        grid_spec=pltpu.PrefetchScalarGridSpec(
            num_scalar_prefetch=2,
            grid=grid,
            in_specs=[
                pl.BlockSpec((_B, h), lambda nb, i, m, t: (nb, 0)),
                pl.BlockSpec((1, h, _IT), lambda nb, i, m, t: (m[nb], 0, i)),
                pl.BlockSpec((1, h, _IT), lambda nb, i, m, t: (m[nb], 0, i)),
                pl.BlockSpec((1, _IT, h), lambda nb, i, m, t: (m[nb], i, 0)),
                pl.BlockSpec((1, 1, _B), lambda nb, i, m, t: (nb, 0, 0)),
            ],
            out_specs=pl.BlockSpec((_B, h), lambda nb, i, m, t: (nb, 0)),
            scratch_shapes=[pltpu.VMEM((_B, h), jnp.float32)],
        ),
        out_shape=jax.ShapeDtypeStruct((p, h), jnp.float32),
    )(block_e, total_nb, xs, wg, wu, wd, aff3)


def kernel(hidden_states, expert_affinities, expert_index, seq_len,
           W_gate, W_up, W_down):
    t, h = hidden_states.shape
    e, _, i_dim = W_gate.shape
    k = expert_index.shape[1]
    tk = t * k
    nb_max = tk // _B + e
    p = nb_max * _B
    ip = pl.cdiv(i_dim, _IT) * _IT

    flat_e = expert_index.reshape(tk).astype(jnp.int32)
    oneh = (flat_e[:, None] == jnp.arange(e, dtype=jnp.int32)[None, :]
            ).astype(jnp.int32)
    counts = oneh.sum(0)
    rank = jnp.take_along_axis(jnp.cumsum(oneh, axis=0) - oneh,
                               flat_e[:, None], axis=1)[:, 0]
    nbe = (counts + _B - 1) // _B
    blk_start = jnp.concatenate(
        [jnp.zeros(1, jnp.int32), jnp.cumsum(nbe).astype(jnp.int32)])
    row_start = blk_start[:e] * _B
    pos = row_start[flat_e] + rank
    total_nb = blk_start[e].reshape(1)
    nb_ids = jnp.arange(nb_max, dtype=jnp.int32)
    block_e = jnp.clip(
        jnp.sum(nb_ids[:, None] >= blk_start[None, :e], axis=1) - 1, 0, e - 1
    ).astype(jnp.int32)

    aff_tk = jnp.take_along_axis(expert_affinities, expert_index, axis=1)
    aff_tk = aff_tk / jnp.sum(aff_tk, axis=-1, keepdims=True)
    valid = (jnp.arange(t) < seq_len).astype(aff_tk.dtype)
    aff_tk = aff_tk * valid[:, None]
    aff_flat = aff_tk.reshape(tk)

    tok_of_pos = jnp.zeros(p, jnp.int32).at[pos].set(
        jnp.arange(tk, dtype=jnp.int32) // k)
    aff_sorted = jnp.zeros(p, jnp.float32).at[pos].set(aff_flat)
    aff3 = aff_sorted.reshape(nb_max, 1, _B)

    xb = hidden_states.astype(jnp.bfloat16)
    pad_i = ip - i_dim
    wg = jnp.pad(W_gate, ((0, 0), (0, 0), (0, pad_i))).astype(jnp.bfloat16)
    wu = jnp.pad(W_up, ((0, 0), (0, 0), (0, pad_i))).astype(jnp.bfloat16)
    wd = jnp.pad(W_down, ((0, 0), (0, pad_i), (0, 0))).astype(jnp.bfloat16)

    xs = xb[tok_of_pos]

    out_sorted = _grouped_glu(xs, wg, wu, wd, aff3, block_e, total_nb)

    pos2 = pos.reshape(t, k)
    out = out_sorted[pos2[:, 0]]
    for kk in range(1, k):
        out = out + out_sorted[pos2[:, kk]]
    return out
